# Initial kernel scaffold; baseline (speedup 1.0000x reference)
import jax
import jax.numpy as jnp
from jax.experimental import pallas as pl


def kernel(src_x, dst_x, edge_index, W_src, b_src, g_src, be_src, W_dst, b_dst, g_dst, be_dst, Wk, bk, Wv, bv, Wq, bq, Wa, ba):
    raise NotImplementedError("write your pallas kernel here")


# speedup vs baseline: 2.1940x; 2.1940x over previous
import functools

import jax
import jax.numpy as jnp
import numpy as np
from jax import lax
from jax.experimental import pallas as pl
from jax.experimental.pallas import tpu as pltpu
from jax.experimental.pallas import tpu_sc as plsc

N_NODES = 10000
E_TOTAL = 320000
D = 128
HID = 128
H = 2
EPS = 1e-5

ROW_BLK = 1000
N_TILES = 16
CHUNK = 64
TOTAL_CHUNKS = E_TOTAL // CHUNK
N_CHUNK_ITERS = -(-TOTAL_CHUNKS // N_TILES)
OUT_ROWS = 624
LAST_BASE = OUT_ROWS * N_TILES
LAST_EXTRA = N_NODES - LAST_BASE
DEN_ROWS = 80


def _stats_body(xs_ref, xd_ref, cs_ref, ms_ref, cd_ref, md_ref):
    @pl.when(pl.program_id(0) == 0)
    def _():
        cs_ref[...] = jnp.zeros_like(cs_ref)
        ms_ref[...] = jnp.zeros_like(ms_ref)
        cd_ref[...] = jnp.zeros_like(cd_ref)
        md_ref[...] = jnp.zeros_like(md_ref)

    xs = xs_ref[...]
    xd = xd_ref[...]
    dn = (((0,), (0,)), ((), ()))
    cs_ref[...] += lax.dot_general(xs, xs, dn, preferred_element_type=jnp.float32)
    ms_ref[...] += jnp.sum(xs, axis=0, keepdims=True)
    cd_ref[...] += lax.dot_general(xd, xd, dn, preferred_element_type=jnp.float32)
    md_ref[...] += jnp.sum(xd, axis=0, keepdims=True)


def _run_stats(src_x, dst_x):
    grid = N_NODES // ROW_BLK
    f32 = jnp.float32
    return pl.pallas_call(
        _stats_body,
        grid=(grid,),
        in_specs=[
            pl.BlockSpec((ROW_BLK, D), lambda i: (i, 0)),
            pl.BlockSpec((ROW_BLK, D), lambda i: (i, 0)),
        ],
        out_specs=[
            pl.BlockSpec((D, D), lambda i: (0, 0)),
            pl.BlockSpec((1, D), lambda i: (0, 0)),
            pl.BlockSpec((D, D), lambda i: (0, 0)),
            pl.BlockSpec((1, D), lambda i: (0, 0)),
        ],
        out_shape=[
            jax.ShapeDtypeStruct((D, D), f32),
            jax.ShapeDtypeStruct((1, D), f32),
            jax.ShapeDtypeStruct((D, D), f32),
            jax.ShapeDtypeStruct((1, D), f32),
        ],
    )(src_x, dst_x)


def _fold_body(cs_ref, ms_ref, cd_ref, md_ref,
               ws_ref, bs_ref, gs_ref, es_ref,
               wd_ref, bd_ref, gd_ref, ed_ref,
               wps_ref, bps_ref, wpd_ref, bpd_ref):
    inv_n = np.float32(1.0 / N_NODES)

    def fold(c_ref, m_ref, w_ref, b_ref, g_ref, e_ref, wo_ref, bo_ref):
        C = c_ref[...]
        m = m_ref[...] * inv_n
        for h in range(H):
            W = w_ref[h]
            g = g_ref[h : h + 1, :]
            be = e_ref[h : h + 1, :]
            mw = jnp.dot(m, W, preferred_element_type=jnp.float32)
            cw = jnp.dot(C, W, preferred_element_type=jnp.float32)
            ex2 = jnp.sum(W * cw, axis=0, keepdims=True) * inv_n
            var = ex2 - mw * mw
            scale = g * lax.rsqrt(var + np.float32(EPS))
            wo_ref[h] = W * scale
            bo_ref[h : h + 1, :] = be - mw * scale

    fold(cs_ref, ms_ref, ws_ref, bs_ref, gs_ref, es_ref, wps_ref, bps_ref)
    fold(cd_ref, md_ref, wd_ref, bd_ref, gd_ref, ed_ref, wpd_ref, bpd_ref)


def _run_fold(cs, ms, cd, md, W_src, b_src, g_src, be_src,
              W_dst, b_dst, g_dst, be_dst):
    f32 = jnp.float32
    full = lambda s: pl.BlockSpec(s, lambda: tuple(0 for _ in s))
    ins = [cs, ms, cd, md, W_src, b_src, g_src, be_src,
           W_dst, b_dst, g_dst, be_dst]
    return pl.pallas_call(
        _fold_body,
        in_specs=[full(x.shape) for x in ins],
        out_specs=[full((H, D, HID)), full((H, HID)),
                   full((H, D, HID)), full((H, HID))],
        out_shape=[
            jax.ShapeDtypeStruct((H, D, HID), f32),
            jax.ShapeDtypeStruct((H, HID), f32),
            jax.ShapeDtypeStruct((H, D, HID), f32),
            jax.ShapeDtypeStruct((H, HID), f32),
        ],
    )(*ins)


def _proj_body(xs_ref, xd_ref, wps_ref, bps_ref, wpd_ref, bpd_ref,
               wk_ref, bk_ref, wv_ref, bv_ref, wq_ref, bq_ref,
               qt_ref, kv_ref, hd_ref):
    xs = xs_ref[...]
    xd = xd_ref[...]
    for h in range(H):
        hs = jnp.maximum(
            jnp.dot(xs, wps_ref[h], preferred_element_type=jnp.float32)
            + bps_ref[h], 0.0)
        hd = jnp.maximum(
            jnp.dot(xd, wpd_ref[h], preferred_element_type=jnp.float32)
            + bpd_ref[h], 0.0)
        k = jnp.dot(hs, wk_ref[h], preferred_element_type=jnp.float32) + bk_ref[h]
        v = jnp.dot(hs, wv_ref[h], preferred_element_type=jnp.float32) + bv_ref[h]
        q = jnp.dot(hd, wq_ref[h], preferred_element_type=jnp.float32) + bq_ref[h]
        qt_ref[h] = q
        kv_ref[h] = jnp.concatenate([k, v], axis=1)
        hd_ref[h] = hd


def _run_proj(src_x, dst_x, wps, bps, wpd, bpd, Wk, bk, Wv, bv, Wq, bq):
    grid = N_NODES // ROW_BLK
    f32 = jnp.float32
    wfull = lambda s: pl.BlockSpec(s, lambda i: tuple(0 for _ in s))
    ins = [wps, bps, wpd, bpd, Wk, bk, Wv, bv, Wq, bq]
    return pl.pallas_call(
        _proj_body,
        grid=(grid,),
        in_specs=[pl.BlockSpec((ROW_BLK, D), lambda i: (i, 0)),
                  pl.BlockSpec((ROW_BLK, D), lambda i: (i, 0))]
                 + [wfull(x.shape) for x in ins],
        out_specs=[
            pl.BlockSpec((H, ROW_BLK, HID), lambda i: (0, i, 0)),
            pl.BlockSpec((H, ROW_BLK, 2 * HID), lambda i: (0, i, 0)),
            pl.BlockSpec((H, ROW_BLK, HID), lambda i: (0, i, 0)),
        ],
        out_shape=[
            jax.ShapeDtypeStruct((H, N_NODES, HID), f32),
            jax.ShapeDtypeStruct((H, N_NODES, 2 * HID), f32),
            jax.ShapeDtypeStruct((H, N_NODES, HID), f32),
        ],
    )(src_x, dst_x, *ins)


def _edge_body(qt_ref, kv_ref, src_ref, dst_ref, zrow_ref,
               t_out_ref, den_out_ref,
               src_i, dst_i, dst_a, dsh, q_rows, kv_rows, vrow, stage,
               t_sh, den_sh, sem1, sem2):
    c = lax.axis_index("c")
    s = lax.axis_index("s")
    off = c * N_NODES
    inv_sqrt = np.float32(1.0 / np.sqrt(HID))
    lane = lax.iota(jnp.int32, 16)
    zero16 = jnp.zeros((16,), jnp.float32)

    pltpu.sync_copy(zrow_ref.at[pl.ds(0, OUT_ROWS)],
                    t_sh.at[pl.ds(s * OUT_ROWS, OUT_ROWS)])

    @pl.when(s == N_TILES - 1)
    def _():
        pltpu.sync_copy(zrow_ref.at[pl.ds(0, LAST_EXTRA)],
                        t_sh.at[pl.ds(LAST_BASE, LAST_EXTRA)])

    @pl.when(s == 0)
    def _():
        pltpu.sync_copy(zrow_ref.at[pl.ds(0, DEN_ROWS)], den_sh)

    for i in range(CHUNK):
        for j in range(HID // 16):
            stage[i, pl.ds(16 * j, 16)] = zero16

    plsc.subcore_barrier()

    def chunk_body(g, carry):
        cid = g * N_TILES + s

        @pl.when(cid < TOTAL_CHUNKS)
        def _():
            base = cid * CHUNK
            pltpu.sync_copy(src_ref.at[pl.ds(base, CHUNK)], src_i)
            pltpu.sync_copy(dst_ref.at[pl.ds(base, CHUNK)], dst_i)
            for j in range(CHUNK // 16):
                sl = pl.ds(16 * j, 16)
                src_i[sl] = src_i[sl] + off
                dst_a[sl] = dst_i[sl] + off
                dsh[sl] = lax.shift_right_logical(dst_i[sl], 7)
            cp1 = pltpu.async_copy(qt_ref.at[dst_a], q_rows, sem1)
            cp2 = pltpu.async_copy(kv_ref.at[src_i], kv_rows, sem2)
            cp1.wait()
            cp2.wait()

            def group_body(gg, cc):
                rows = lane + gg * 16
                dvec = zero16
                for j in range(HID):
                    cj = jnp.full((16,), j, jnp.int32)
                    qv = plsc.load_gather(q_rows, [rows, cj])
                    kj = plsc.load_gather(kv_rows, [rows, cj])
                    dvec = dvec + qv * kj
                x = dvec * inv_sqrt
                sg = 1.0 / (1.0 + jnp.exp(-x))
                svec = jnp.exp(sg)
                for j in range(HID):
                    cj = jnp.full((16,), j, jnp.int32)
                    cv = jnp.full((16,), HID + j, jnp.int32)
                    vv = plsc.load_gather(kv_rows, [rows, cv])
                    plsc.store_scatter(vrow, [rows, cj], vv * svec)
                dst_g = dst_i[pl.ds(gg * 16, 16)]
                lpos = lax.bitwise_and(dst_g,
                                       jnp.full((16,), 127, jnp.int32))
                plsc.store_scatter(stage, [rows, lpos], svec)
                return cc

            lax.fori_loop(0, CHUNK // 16, group_body, 0)
            pltpu.sync_copy(vrow, t_sh.at[dst_i], add=True)
            pltpu.sync_copy(stage, den_sh.at[dsh], add=True)

            def clear_body(gg, cc):
                rows = lane + gg * 16
                dst_g = dst_i[pl.ds(gg * 16, 16)]
                lpos = lax.bitwise_and(dst_g,
                                       jnp.full((16,), 127, jnp.int32))
                plsc.store_scatter(stage, [rows, lpos], zero16)
                return cc

            lax.fori_loop(0, CHUNK // 16, clear_body, 0)

        return carry

    lax.fori_loop(0, N_CHUNK_ITERS, chunk_body, 0)
    plsc.subcore_barrier()

    ob = c * N_NODES + s * OUT_ROWS
    pltpu.sync_copy(t_sh.at[pl.ds(s * OUT_ROWS, OUT_ROWS)],
                    t_out_ref.at[pl.ds(ob, OUT_ROWS)])

    @pl.when(s == N_TILES - 1)
    def _():
        pltpu.sync_copy(t_sh.at[pl.ds(LAST_BASE, LAST_EXTRA)],
                        t_out_ref.at[pl.ds(c * N_NODES + LAST_BASE,
                                           LAST_EXTRA)])

    @pl.when(s < DEN_ROWS // 8)
    def _():
        pltpu.sync_copy(den_sh.at[pl.ds(s * 8, 8)],
                        den_out_ref.at[pl.ds(c * DEN_ROWS + s * 8, 8)])


def _run_edge(qt2, kv2, src_e, dst_e, zrow):
    f32 = jnp.float32
    mesh = plsc.VectorSubcoreMesh(core_axis_name="c", subcore_axis_name="s")
    kfn = functools.partial(
        pl.kernel,
        mesh=mesh,
        compiler_params=pltpu.CompilerParams(needs_layout_passes=False),
        out_type=[
            jax.ShapeDtypeStruct((H * N_NODES, HID), f32),
            jax.ShapeDtypeStruct((H * DEN_ROWS, HID), f32),
        ],
        scratch_types=[
            pltpu.VMEM((CHUNK,), jnp.int32),
            pltpu.VMEM((CHUNK,), jnp.int32),
            pltpu.VMEM((CHUNK,), jnp.int32),
            pltpu.VMEM((CHUNK,), jnp.int32),
            pltpu.VMEM((CHUNK, HID), f32),
            pltpu.VMEM((CHUNK, 2 * HID), f32),
            pltpu.VMEM((CHUNK, HID), f32),
            pltpu.VMEM((CHUNK, HID), f32),
            pltpu.VMEM_SHARED((N_NODES, HID), f32),
            pltpu.VMEM_SHARED((DEN_ROWS, HID), f32),
            pltpu.SemaphoreType.DMA,
            pltpu.SemaphoreType.DMA,
        ],
    )
    return kfn(_edge_body)(qt2, kv2, src_e, dst_e, zrow)


def _out_body(t_ref, den_ref, hd_ref, wa_ref, ba_ref, out_ref):
    acc = None
    for h in range(H):
        den = den_ref[h]
        den = jnp.where(den == 0.0, np.float32(1.0), den)
        trans = t_ref[h] / den + hd_ref[h]
        o = jnp.dot(trans, wa_ref[h], preferred_element_type=jnp.float32) \
            + ba_ref[h]
        acc = o if acc is None else acc + o
    out_ref[...] = np.float32(1.0 / H) * acc


def _run_out(t, den, hd, Wa, ba):
    grid = N_NODES // ROW_BLK
    f32 = jnp.float32
    wfull = lambda s: pl.BlockSpec(s, lambda i: tuple(0 for _ in s))
    return pl.pallas_call(
        _out_body,
        grid=(grid,),
        in_specs=[
            pl.BlockSpec((H, ROW_BLK, HID), lambda i: (0, i, 0)),
            pl.BlockSpec((H, ROW_BLK, 1), lambda i: (0, i, 0)),
            pl.BlockSpec((H, ROW_BLK, HID), lambda i: (0, i, 0)),
            wfull(Wa.shape),
            wfull(ba.shape),
        ],
        out_specs=pl.BlockSpec((ROW_BLK, D), lambda i: (i, 0)),
        out_shape=jax.ShapeDtypeStruct((N_NODES, D), f32),
    )(t, den, hd, Wa, ba)


def kernel(src_x, dst_x, edge_index, W_src, b_src, g_src, be_src,
           W_dst, b_dst, g_dst, be_dst, Wk, bk, Wv, bv, Wq, bq, Wa, ba):
    src_e = edge_index[0].astype(jnp.int32)
    dst_e = edge_index[1].astype(jnp.int32)

    cs, ms, cd, md = _run_stats(src_x, dst_x)
    wps, bps, wpd, bpd = _run_fold(cs, ms, cd, md,
                                   W_src, b_src, g_src, be_src,
                                   W_dst, b_dst, g_dst, be_dst)
    qt, kv, hd = _run_proj(src_x, dst_x, wps, bps, wpd, bpd,
                           Wk, bk, Wv, bv, Wq, bq)
    qt2 = qt.reshape(H * N_NODES, HID)
    kv2 = kv.reshape(H * N_NODES, 2 * HID)
    zrow = jnp.zeros((OUT_ROWS, HID), jnp.float32)
    t_out, den_out = _run_edge(qt2, kv2, src_e, dst_e, zrow)
    t = t_out.reshape(H, N_NODES, HID)
    den = den_out.reshape(H, DEN_ROWS * HID)[:, :N_NODES]
    den = den.reshape(H, N_NODES, 1)
    return _run_out(t, den, hd, Wa, ba)


# speedup vs baseline: 2.5569x; 1.1654x over previous
import functools

import jax
import jax.numpy as jnp
import numpy as np
from jax import lax
from jax.experimental import pallas as pl
from jax.experimental.pallas import tpu as pltpu
from jax.experimental.pallas import tpu_sc as plsc

N_NODES = 10000
E_TOTAL = 320000
D = 128
HID = 128
H = 2
EPS = 1e-5

ROW_BLK = 1000
N_TILES = 16
CHUNK = 32
EDGES_PER_TILE = E_TOTAL // N_TILES
CHUNKS_PER_TILE = EDGES_PER_TILE // CHUNK
SUPER = 800
CH_PER_SUP = SUPER // CHUNK
OUT_ROWS = 624
LAST_BASE = OUT_ROWS * N_TILES
LAST_EXTRA = N_NODES - LAST_BASE
DEN_ROWS = 80


def _stats_body(xs_ref, xd_ref, cs_ref, ms_ref, cd_ref, md_ref):
    @pl.when(pl.program_id(0) == 0)
    def _():
        cs_ref[...] = jnp.zeros_like(cs_ref)
        ms_ref[...] = jnp.zeros_like(ms_ref)
        cd_ref[...] = jnp.zeros_like(cd_ref)
        md_ref[...] = jnp.zeros_like(md_ref)

    xs = xs_ref[...]
    xd = xd_ref[...]
    dn = (((0,), (0,)), ((), ()))
    cs_ref[...] += lax.dot_general(xs, xs, dn, preferred_element_type=jnp.float32)
    ms_ref[...] += jnp.sum(xs, axis=0, keepdims=True)
    cd_ref[...] += lax.dot_general(xd, xd, dn, preferred_element_type=jnp.float32)
    md_ref[...] += jnp.sum(xd, axis=0, keepdims=True)


def _run_stats(src_x, dst_x):
    grid = N_NODES // ROW_BLK
    f32 = jnp.float32
    return pl.pallas_call(
        _stats_body,
        grid=(grid,),
        in_specs=[
            pl.BlockSpec((ROW_BLK, D), lambda i: (i, 0)),
            pl.BlockSpec((ROW_BLK, D), lambda i: (i, 0)),
        ],
        out_specs=[
            pl.BlockSpec((D, D), lambda i: (0, 0)),
            pl.BlockSpec((1, D), lambda i: (0, 0)),
            pl.BlockSpec((D, D), lambda i: (0, 0)),
            pl.BlockSpec((1, D), lambda i: (0, 0)),
        ],
        out_shape=[
            jax.ShapeDtypeStruct((D, D), f32),
            jax.ShapeDtypeStruct((1, D), f32),
            jax.ShapeDtypeStruct((D, D), f32),
            jax.ShapeDtypeStruct((1, D), f32),
        ],
    )(src_x, dst_x)


def _fold_body(cs_ref, ms_ref, cd_ref, md_ref,
               ws_ref, bs_ref, gs_ref, es_ref,
               wd_ref, bd_ref, gd_ref, ed_ref,
               wps_ref, bps_ref, wpd_ref, bpd_ref):
    inv_n = np.float32(1.0 / N_NODES)

    def fold(c_ref, m_ref, w_ref, b_ref, g_ref, e_ref, wo_ref, bo_ref):
        C = c_ref[...]
        m = m_ref[...] * inv_n
        for h in range(H):
            W = w_ref[h]
            g = g_ref[h : h + 1, :]
            be = e_ref[h : h + 1, :]
            mw = jnp.dot(m, W, preferred_element_type=jnp.float32)
            cw = jnp.dot(C, W, preferred_element_type=jnp.float32)
            ex2 = jnp.sum(W * cw, axis=0, keepdims=True) * inv_n
            var = ex2 - mw * mw
            scale = g * lax.rsqrt(var + np.float32(EPS))
            wo_ref[h] = W * scale
            bo_ref[h : h + 1, :] = be - mw * scale

    fold(cs_ref, ms_ref, ws_ref, bs_ref, gs_ref, es_ref, wps_ref, bps_ref)
    fold(cd_ref, md_ref, wd_ref, bd_ref, gd_ref, ed_ref, wpd_ref, bpd_ref)


def _run_fold(cs, ms, cd, md, W_src, b_src, g_src, be_src,
              W_dst, b_dst, g_dst, be_dst):
    f32 = jnp.float32
    full = lambda s: pl.BlockSpec(s, lambda: tuple(0 for _ in s))
    ins = [cs, ms, cd, md, W_src, b_src, g_src, be_src,
           W_dst, b_dst, g_dst, be_dst]
    return pl.pallas_call(
        _fold_body,
        in_specs=[full(x.shape) for x in ins],
        out_specs=[full((H, D, HID)), full((H, HID)),
                   full((H, D, HID)), full((H, HID))],
        out_shape=[
            jax.ShapeDtypeStruct((H, D, HID), f32),
            jax.ShapeDtypeStruct((H, HID), f32),
            jax.ShapeDtypeStruct((H, D, HID), f32),
            jax.ShapeDtypeStruct((H, HID), f32),
        ],
    )(*ins)


def _proj_body(xs_ref, xd_ref, wps_ref, bps_ref, wpd_ref, bpd_ref,
               wk_ref, bk_ref, wv_ref, bv_ref, wq_ref, bq_ref,
               qt_ref, kv_ref, hd_ref):
    xs = xs_ref[...]
    xd = xd_ref[...]
    for h in range(H):
        hs = jnp.maximum(
            jnp.dot(xs, wps_ref[h], preferred_element_type=jnp.float32)
            + bps_ref[h], 0.0)
        hd = jnp.maximum(
            jnp.dot(xd, wpd_ref[h], preferred_element_type=jnp.float32)
            + bpd_ref[h], 0.0)
        k = jnp.dot(hs, wk_ref[h], preferred_element_type=jnp.float32) + bk_ref[h]
        v = jnp.dot(hs, wv_ref[h], preferred_element_type=jnp.float32) + bv_ref[h]
        q = jnp.dot(hd, wq_ref[h], preferred_element_type=jnp.float32) + bq_ref[h]
        qt_ref[h] = q
        kv_ref[h] = jnp.concatenate([k, v], axis=1)
        hd_ref[h] = hd


def _run_proj(src_x, dst_x, wps, bps, wpd, bpd, Wk, bk, Wv, bv, Wq, bq):
    grid = N_NODES // ROW_BLK
    f32 = jnp.float32
    wfull = lambda s: pl.BlockSpec(s, lambda i: tuple(0 for _ in s))
    ins = [wps, bps, wpd, bpd, Wk, bk, Wv, bv, Wq, bq]
    return pl.pallas_call(
        _proj_body,
        grid=(grid,),
        in_specs=[pl.BlockSpec((ROW_BLK, D), lambda i: (i, 0)),
                  pl.BlockSpec((ROW_BLK, D), lambda i: (i, 0))]
                 + [wfull(x.shape) for x in ins],
        out_specs=[
            pl.BlockSpec((H, ROW_BLK, HID), lambda i: (0, i, 0)),
            pl.BlockSpec((H, ROW_BLK, 2 * HID), lambda i: (0, i, 0)),
            pl.BlockSpec((H, ROW_BLK, HID), lambda i: (0, i, 0)),
        ],
        out_shape=[
            jax.ShapeDtypeStruct((H, N_NODES, HID), f32),
            jax.ShapeDtypeStruct((H, N_NODES, 2 * HID), f32),
            jax.ShapeDtypeStruct((H, N_NODES, HID), f32),
        ],
    )(src_x, dst_x, *ins)


def _edge_body(qt_ref, kv_ref, src_ref, dst_ref, zrow_ref,
               t_out_ref, den_out_ref,
               src_sup, dst_sup,
               src_a0, src_a1, dst_a0, dst_a1, dst_r0, dst_r1,
               dsh0, dsh1, lpos0, lpos1, dst_s0, dst_s1, dsh_s0, dsh_s1,
               q0, q1, kv0, kv1, vr0, vr1, st0, st1,
               t_sh, den_sh,
               semq0, semq1, semk0, semk1, semv0, semv1, sems0, sems1):
    c = lax.axis_index("c")
    s = lax.axis_index("s")
    off = c * N_NODES
    tile_base = s * EDGES_PER_TILE
    inv_sqrt = np.float32(1.0 / np.sqrt(HID))
    lane = lax.iota(jnp.int32, 16)
    zero16 = jnp.zeros((16,), jnp.float32)
    c127 = jnp.full((16,), 127, jnp.int32)

    bufs = (
        (src_a0, dst_a0, dst_r0, dsh0, lpos0, dst_s0, dsh_s0,
         q0, kv0, vr0, st0, semq0, semk0, semv0, sems0),
        (src_a1, dst_a1, dst_r1, dsh1, lpos1, dst_s1, dsh_s1,
         q1, kv1, vr1, st1, semq1, semk1, semv1, sems1),
    )

    pltpu.sync_copy(zrow_ref.at[pl.ds(0, OUT_ROWS)],
                    t_sh.at[pl.ds(s * OUT_ROWS, OUT_ROWS)])

    @pl.when(s == N_TILES - 1)
    def _():
        pltpu.sync_copy(zrow_ref.at[pl.ds(0, LAST_EXTRA)],
                        t_sh.at[pl.ds(LAST_BASE, LAST_EXTRA)])

    @pl.when(s == 0)
    def _():
        pltpu.sync_copy(zrow_ref.at[pl.ds(0, DEN_ROWS)], den_sh)

    for st in (st0, st1):
        for i in range(CHUNK):
            for j in range(HID // 16):
                st[i, pl.ds(16 * j, 16)] = zero16

    plsc.subcore_barrier()

    def load_super(gn):
        @pl.when(lax.rem(gn, CH_PER_SUP) == 0)
        def _():
            base = tile_base + gn * CHUNK
            pltpu.sync_copy(src_ref.at[pl.ds(base, SUPER)], src_sup)
            pltpu.sync_copy(dst_ref.at[pl.ds(base, SUPER)], dst_sup)

    def prep_and_fire(gn, b):
        src_a, dst_a, dst_r, dsh, _, _, _, q_rows, kv_rows, _, _, \
            semq, semk, _, _ = bufs[b]
        pos = lax.rem(gn, CH_PER_SUP) * CHUNK
        for j in range(CHUNK // 16):
            sl = pl.ds(pos + 16 * j, 16)
            ob = pl.ds(16 * j, 16)
            sv = src_sup[sl]
            dv = dst_sup[sl]
            src_a[ob] = sv + off
            dst_a[ob] = dv + off
            dst_r[ob] = dv
            dsh[ob] = lax.shift_right_logical(dv, 7)
        pltpu.async_copy(qt_ref.at[dst_a], q_rows, semq)
        pltpu.async_copy(kv_ref.at[src_a], kv_rows, semk)

    def wait_gathers(b):
        src_a, dst_a, _, _, _, _, _, q_rows, kv_rows, _, _, \
            semq, semk, _, _ = bufs[b]
        pltpu.make_async_copy(qt_ref.at[dst_a], q_rows, semq).wait()
        pltpu.make_async_copy(kv_ref.at[src_a], kv_rows, semk).wait()

    def wait_scatters(b):
        _, _, _, _, _, dst_s, dsh_s, _, _, vrow, stage, \
            _, _, semv, sems = bufs[b]
        pltpu.make_async_copy(vrow, t_sh.at[dst_s], semv).wait()
        pltpu.make_async_copy(stage, den_sh.at[dsh_s], sems).wait()

    def clear_stage(b):
        _, _, _, _, lpos, _, _, _, _, _, stage, _, _, _, _ = bufs[b]

        def cbody(gg, cc):
            rows = lane + gg * 16
            lp = lpos[pl.ds(gg * 16, 16)]
            plsc.store_scatter(stage, [rows, lp], zero16)
            return cc

        lax.fori_loop(0, CHUNK // 16, cbody, 0)

    def compute_and_fire(b):
        _, _, dst_r, dsh, lpos, dst_s, dsh_s, q_rows, kv_rows, vrow, \
            stage, _, _, semv, sems = bufs[b]

        def gbody(gg, cc):
            rows = lane + gg * 16
            zi16 = jnp.zeros((16,), jnp.int32)

            def dot_step(j, car):
                dvec, cj = car
                qv = plsc.load_gather(q_rows, [rows, cj])
                kj = plsc.load_gather(kv_rows, [rows, cj])
                return (dvec + qv * kj, cj + 1)

            dvec, _ = lax.fori_loop(0, HID, dot_step, (zero16, zi16),
                                    unroll=16)
            x = dvec * inv_sqrt
            sg = 1.0 / (1.0 + jnp.exp(-x))
            svec = jnp.exp(sg)

            def sc_step(j, car):
                cv, cj = car
                vv = plsc.load_gather(kv_rows, [rows, cv])
                plsc.store_scatter(vrow, [rows, cj], vv * svec)
                return (cv + 1, cj + 1)

            lax.fori_loop(0, HID, sc_step,
                          (jnp.full((16,), HID, jnp.int32), zi16),
                          unroll=16)
            dst_g = dst_r[pl.ds(gg * 16, 16)]
            lp = lax.bitwise_and(dst_g, c127)
            lpos[pl.ds(gg * 16, 16)] = lp
            plsc.store_scatter(stage, [rows, lp], svec)
            return cc

        lax.fori_loop(0, CHUNK // 16, gbody, 0)
        for j in range(CHUNK // 16):
            sl = pl.ds(16 * j, 16)
            dst_s[sl] = dst_r[sl]
            dsh_s[sl] = dsh[sl]
        pltpu.async_copy(vrow, t_sh.at[dst_s], semv, add=True)
        pltpu.async_copy(stage, den_sh.at[dsh_s], sems, add=True)

    load_super(jnp.int32(0))
    prep_and_fire(jnp.int32(0), 0)

    def pair_body(t, carry):
        for half in range(2):
            g = t * 2 + half
            b = half
            gn = g + 1
            load_super(gn)
            prep_and_fire(gn, 1 - b)

            @pl.when(t >= 1)
            def _():
                wait_scatters(b)
                clear_stage(b)

            wait_gathers(b)
            compute_and_fire(b)
        return carry

    lax.fori_loop(0, (CHUNKS_PER_TILE - 1) // 2, pair_body, 0)

    wait_scatters(0)
    clear_stage(0)
    wait_gathers(0)
    compute_and_fire(0)

    wait_scatters(1)
    wait_scatters(0)
    plsc.subcore_barrier()

    ob = c * N_NODES + s * OUT_ROWS
    pltpu.sync_copy(t_sh.at[pl.ds(s * OUT_ROWS, OUT_ROWS)],
                    t_out_ref.at[pl.ds(ob, OUT_ROWS)])

    @pl.when(s == N_TILES - 1)
    def _():
        pltpu.sync_copy(t_sh.at[pl.ds(LAST_BASE, LAST_EXTRA)],
                        t_out_ref.at[pl.ds(c * N_NODES + LAST_BASE,
                                           LAST_EXTRA)])

    @pl.when(s < DEN_ROWS // 8)
    def _():
        pltpu.sync_copy(den_sh.at[pl.ds(s * 8, 8)],
                        den_out_ref.at[pl.ds(c * DEN_ROWS + s * 8, 8)])


def _run_edge(qt2, kv2, src_e, dst_e, zrow):
    f32 = jnp.float32
    mesh = plsc.VectorSubcoreMesh(core_axis_name="c", subcore_axis_name="s")
    kfn = functools.partial(
        pl.kernel,
        mesh=mesh,
        compiler_params=pltpu.CompilerParams(needs_layout_passes=False),
        out_type=[
            jax.ShapeDtypeStruct((H * N_NODES, HID), f32),
            jax.ShapeDtypeStruct((H * DEN_ROWS, HID), f32),
        ],
        scratch_types=(
            [pltpu.VMEM((SUPER,), jnp.int32)] * 2
            + [pltpu.VMEM((CHUNK,), jnp.int32)] * 14
            + [pltpu.VMEM((CHUNK, HID), f32),
               pltpu.VMEM((CHUNK, HID), f32),
               pltpu.VMEM((CHUNK, 2 * HID), f32),
               pltpu.VMEM((CHUNK, 2 * HID), f32),
               pltpu.VMEM((CHUNK, HID), f32),
               pltpu.VMEM((CHUNK, HID), f32),
               pltpu.VMEM((CHUNK, HID), f32),
               pltpu.VMEM((CHUNK, HID), f32),
               pltpu.VMEM_SHARED((N_NODES, HID), f32),
               pltpu.VMEM_SHARED((DEN_ROWS, HID), f32)]
            + [pltpu.SemaphoreType.DMA] * 8
        ),
    )
    return kfn(_edge_body)(qt2, kv2, src_e, dst_e, zrow)


def _out_body(t_ref, den_ref, hd_ref, wa_ref, ba_ref, out_ref):
    acc = None
    for h in range(H):
        den = den_ref[h]
        den = jnp.where(den == 0.0, np.float32(1.0), den)
        trans = t_ref[h] / den + hd_ref[h]
        o = jnp.dot(trans, wa_ref[h], preferred_element_type=jnp.float32) \
            + ba_ref[h]
        acc = o if acc is None else acc + o
    out_ref[...] = np.float32(1.0 / H) * acc


def _run_out(t, den, hd, Wa, ba):
    grid = N_NODES // ROW_BLK
    f32 = jnp.float32
    wfull = lambda s: pl.BlockSpec(s, lambda i: tuple(0 for _ in s))
    return pl.pallas_call(
        _out_body,
        grid=(grid,),
        in_specs=[
            pl.BlockSpec((H, ROW_BLK, HID), lambda i: (0, i, 0)),
            pl.BlockSpec((H, ROW_BLK, 1), lambda i: (0, i, 0)),
            pl.BlockSpec((H, ROW_BLK, HID), lambda i: (0, i, 0)),
            wfull(Wa.shape),
            wfull(ba.shape),
        ],
        out_specs=pl.BlockSpec((ROW_BLK, D), lambda i: (i, 0)),
        out_shape=jax.ShapeDtypeStruct((N_NODES, D), f32),
    )(t, den, hd, Wa, ba)


def kernel(src_x, dst_x, edge_index, W_src, b_src, g_src, be_src,
           W_dst, b_dst, g_dst, be_dst, Wk, bk, Wv, bv, Wq, bq, Wa, ba):
    src_e = edge_index[0].astype(jnp.int32)
    dst_e = edge_index[1].astype(jnp.int32)

    cs, ms, cd, md = _run_stats(src_x, dst_x)
    wps, bps, wpd, bpd = _run_fold(cs, ms, cd, md,
                                   W_src, b_src, g_src, be_src,
                                   W_dst, b_dst, g_dst, be_dst)
    qt, kv, hd = _run_proj(src_x, dst_x, wps, bps, wpd, bpd,
                           Wk, bk, Wv, bv, Wq, bq)
    qt2 = qt.reshape(H * N_NODES, HID)
    kv2 = kv.reshape(H * N_NODES, 2 * HID)
    zrow = jnp.zeros((OUT_ROWS, HID), jnp.float32)
    t_out, den_out = _run_edge(qt2, kv2, src_e, dst_e, zrow)
    t = t_out.reshape(H, N_NODES, HID)
    den = den_out.reshape(H, DEN_ROWS * HID)[:, :N_NODES]
    den = den.reshape(H, N_NODES, 1)
    return _run_out(t, den, hd, Wa, ba)


# speedup vs baseline: 3.6405x; 1.4238x over previous
import functools

import jax
import jax.numpy as jnp
import numpy as np
from jax import lax
from jax.experimental import pallas as pl
from jax.experimental.pallas import tpu as pltpu
from jax.experimental.pallas import tpu_sc as plsc

N_NODES = 10000
E_TOTAL = 320000
D = 128
HID = 128
H = 2
EPS = 1e-5

ROW_BLK = 2000
N_TILES = 16
CHUNK = 32
EDGES_PER_TILE = E_TOTAL // N_TILES
CHUNKS_PER_TILE = EDGES_PER_TILE // CHUNK
SUPER = 800
CH_PER_SUP = SUPER // CHUNK
OUT_ROWS = 624
LAST_BASE = OUT_ROWS * N_TILES
LAST_EXTRA = N_NODES - LAST_BASE
DEN_ROWS = 80


def _stats_body(xs_ref, xd_ref, cs_ref, ms_ref, cd_ref, md_ref):
    @pl.when(pl.program_id(0) == 0)
    def _():
        cs_ref[...] = jnp.zeros_like(cs_ref)
        ms_ref[...] = jnp.zeros_like(ms_ref)
        cd_ref[...] = jnp.zeros_like(cd_ref)
        md_ref[...] = jnp.zeros_like(md_ref)

    xs = xs_ref[...]
    xd = xd_ref[...]
    dn = (((0,), (0,)), ((), ()))
    cs_ref[...] += lax.dot_general(xs, xs, dn, preferred_element_type=jnp.float32)
    ms_ref[...] += jnp.sum(xs, axis=0, keepdims=True)
    cd_ref[...] += lax.dot_general(xd, xd, dn, preferred_element_type=jnp.float32)
    md_ref[...] += jnp.sum(xd, axis=0, keepdims=True)


def _run_stats(src_x, dst_x):
    grid = N_NODES // ROW_BLK
    f32 = jnp.float32
    return pl.pallas_call(
        _stats_body,
        grid=(grid,),
        in_specs=[
            pl.BlockSpec((ROW_BLK, D), lambda i: (i, 0)),
            pl.BlockSpec((ROW_BLK, D), lambda i: (i, 0)),
        ],
        out_specs=[
            pl.BlockSpec((D, D), lambda i: (0, 0)),
            pl.BlockSpec((1, D), lambda i: (0, 0)),
            pl.BlockSpec((D, D), lambda i: (0, 0)),
            pl.BlockSpec((1, D), lambda i: (0, 0)),
        ],
        out_shape=[
            jax.ShapeDtypeStruct((D, D), f32),
            jax.ShapeDtypeStruct((1, D), f32),
            jax.ShapeDtypeStruct((D, D), f32),
            jax.ShapeDtypeStruct((1, D), f32),
        ],
    )(src_x, dst_x)


def _fold_body(cs_ref, ms_ref, cd_ref, md_ref,
               ws_ref, bs_ref, gs_ref, es_ref,
               wd_ref, bd_ref, gd_ref, ed_ref,
               wps_ref, bps_ref, wpd_ref, bpd_ref):
    inv_n = np.float32(1.0 / N_NODES)

    def fold(c_ref, m_ref, w_ref, b_ref, g_ref, e_ref, wo_ref, bo_ref):
        C = c_ref[...]
        m = m_ref[...] * inv_n
        for h in range(H):
            W = w_ref[h]
            g = g_ref[h : h + 1, :]
            be = e_ref[h : h + 1, :]
            mw = jnp.dot(m, W, preferred_element_type=jnp.float32)
            cw = jnp.dot(C, W, preferred_element_type=jnp.float32)
            ex2 = jnp.sum(W * cw, axis=0, keepdims=True) * inv_n
            var = ex2 - mw * mw
            scale = g * lax.rsqrt(var + np.float32(EPS))
            wo_ref[h] = W * scale
            bo_ref[h : h + 1, :] = be - mw * scale

    fold(cs_ref, ms_ref, ws_ref, bs_ref, gs_ref, es_ref, wps_ref, bps_ref)
    fold(cd_ref, md_ref, wd_ref, bd_ref, gd_ref, ed_ref, wpd_ref, bpd_ref)


def _run_fold(cs, ms, cd, md, W_src, b_src, g_src, be_src,
              W_dst, b_dst, g_dst, be_dst):
    f32 = jnp.float32
    full = lambda s: pl.BlockSpec(s, lambda: tuple(0 for _ in s))
    ins = [cs, ms, cd, md, W_src, b_src, g_src, be_src,
           W_dst, b_dst, g_dst, be_dst]
    return pl.pallas_call(
        _fold_body,
        in_specs=[full(x.shape) for x in ins],
        out_specs=[full((H, D, HID)), full((H, HID)),
                   full((H, D, HID)), full((H, HID))],
        out_shape=[
            jax.ShapeDtypeStruct((H, D, HID), f32),
            jax.ShapeDtypeStruct((H, HID), f32),
            jax.ShapeDtypeStruct((H, D, HID), f32),
            jax.ShapeDtypeStruct((H, HID), f32),
        ],
    )(*ins)


def _proj_body(xs_ref, xd_ref, wps_ref, bps_ref, wpd_ref, bpd_ref,
               wk_ref, bk_ref, wv_ref, bv_ref, wq_ref, bq_ref,
               qt_ref, kv_ref, hd_ref):
    xs = xs_ref[...]
    xd = xd_ref[...]
    for h in range(H):
        hs = jnp.maximum(
            jnp.dot(xs, wps_ref[h], preferred_element_type=jnp.float32)
            + bps_ref[h], 0.0)
        hd = jnp.maximum(
            jnp.dot(xd, wpd_ref[h], preferred_element_type=jnp.float32)
            + bpd_ref[h], 0.0)
        k = jnp.dot(hs, wk_ref[h], preferred_element_type=jnp.float32) + bk_ref[h]
        v = jnp.dot(hs, wv_ref[h], preferred_element_type=jnp.float32) + bv_ref[h]
        q = jnp.dot(hd, wq_ref[h], preferred_element_type=jnp.float32) + bq_ref[h]
        qt_ref[h] = q
        kv_ref[h] = jnp.concatenate([k, v], axis=1).astype(jnp.bfloat16)
        hd_ref[h] = hd


def _run_proj(src_x, dst_x, wps, bps, wpd, bpd, Wk, bk, Wv, bv, Wq, bq):
    grid = N_NODES // ROW_BLK
    f32 = jnp.float32
    wfull = lambda s: pl.BlockSpec(s, lambda i: tuple(0 for _ in s))
    ins = [wps, bps, wpd, bpd, Wk, bk, Wv, bv, Wq, bq]
    return pl.pallas_call(
        _proj_body,
        grid=(grid,),
        in_specs=[pl.BlockSpec((ROW_BLK, D), lambda i: (i, 0)),
                  pl.BlockSpec((ROW_BLK, D), lambda i: (i, 0))]
                 + [wfull(x.shape) for x in ins],
        out_specs=[
            pl.BlockSpec((H, ROW_BLK, HID), lambda i: (0, i, 0)),
            pl.BlockSpec((H, ROW_BLK, 2 * HID), lambda i: (0, i, 0)),
            pl.BlockSpec((H, ROW_BLK, HID), lambda i: (0, i, 0)),
        ],
        out_shape=[
            jax.ShapeDtypeStruct((H, N_NODES, HID), f32),
            jax.ShapeDtypeStruct((H, N_NODES, 2 * HID), jnp.bfloat16),
            jax.ShapeDtypeStruct((H, N_NODES, HID), f32),
        ],
    )(src_x, dst_x, *ins)


def _edge_body(qt_ref, kv_ref, src_ref, dst_ref, zrow_ref,
               t_out_ref, den_out_ref,
               src_sup, dst_sup,
               src_a0, src_a1, dst_a0, dst_a1, dst_r0, dst_r1,
               dsh0, dsh1, lpos0, lpos1, dst_s0, dst_s1, dsh_s0, dsh_s1,
               q0, q1, kv0, kv1, vr0, vr1, st0, st1,
               t_sh, den_sh,
               semq0, semq1, semk0, semk1, semv0, semv1, sems0, sems1):
    c = lax.axis_index("c")
    s = lax.axis_index("s")
    off = c * N_NODES
    tile_base = s * EDGES_PER_TILE
    inv_sqrt = np.float32(1.0 / np.sqrt(HID))
    lane = lax.iota(jnp.int32, 16)
    zero16 = jnp.zeros((16,), jnp.float32)
    c127 = jnp.full((16,), 127, jnp.int32)

    bufs = (
        (src_a0, dst_a0, dst_r0, dsh0, lpos0, dst_s0, dsh_s0,
         q0, kv0, vr0, st0, semq0, semk0, semv0, sems0),
        (src_a1, dst_a1, dst_r1, dsh1, lpos1, dst_s1, dsh_s1,
         q1, kv1, vr1, st1, semq1, semk1, semv1, sems1),
    )

    pltpu.sync_copy(zrow_ref.at[pl.ds(0, OUT_ROWS)],
                    t_sh.at[pl.ds(s * OUT_ROWS, OUT_ROWS)])

    @pl.when(s == N_TILES - 1)
    def _():
        pltpu.sync_copy(zrow_ref.at[pl.ds(0, LAST_EXTRA)],
                        t_sh.at[pl.ds(LAST_BASE, LAST_EXTRA)])

    @pl.when(s == 0)
    def _():
        pltpu.sync_copy(zrow_ref.at[pl.ds(0, DEN_ROWS)], den_sh)

    for st in (st0, st1):
        for i in range(CHUNK):
            for j in range(HID // 16):
                st[i, pl.ds(16 * j, 16)] = zero16

    plsc.subcore_barrier()

    def load_super(gn):
        @pl.when(lax.rem(gn, CH_PER_SUP) == 0)
        def _():
            base = tile_base + gn * CHUNK
            pltpu.sync_copy(src_ref.at[pl.ds(base, SUPER)], src_sup)
            pltpu.sync_copy(dst_ref.at[pl.ds(base, SUPER)], dst_sup)

    def prep_and_fire(gn, b):
        src_a, dst_a, dst_r, dsh, _, _, _, q_rows, kv_rows, _, _, \
            semq, semk, _, _ = bufs[b]
        pos = lax.rem(gn, CH_PER_SUP) * CHUNK
        for j in range(CHUNK // 16):
            sl = pl.ds(pos + 16 * j, 16)
            ob = pl.ds(16 * j, 16)
            sv = src_sup[sl]
            dv = dst_sup[sl]
            src_a[ob] = sv + off
            dst_a[ob] = dv + off
            dst_r[ob] = dv
            dsh[ob] = lax.shift_right_logical(dv, 7)
        pltpu.async_copy(qt_ref.at[dst_a], q_rows, semq)
        pltpu.async_copy(kv_ref.at[src_a], kv_rows, semk)

    def wait_gathers(b):
        src_a, dst_a, _, _, _, _, _, q_rows, kv_rows, _, _, \
            semq, semk, _, _ = bufs[b]
        pltpu.make_async_copy(qt_ref.at[dst_a], q_rows, semq).wait()
        pltpu.make_async_copy(kv_ref.at[src_a], kv_rows, semk).wait()

    def wait_scatters(b):
        _, _, _, _, _, dst_s, dsh_s, _, _, vrow, stage, \
            _, _, semv, sems = bufs[b]
        pltpu.make_async_copy(vrow, t_sh.at[dst_s], semv).wait()
        pltpu.make_async_copy(stage, den_sh.at[dsh_s], sems).wait()

    def clear_stage(b):
        _, _, _, _, lpos, _, _, _, _, _, stage, _, _, _, _ = bufs[b]

        def cbody(gg, cc):
            rows = lane + gg * 16
            lp = lpos[pl.ds(gg * 16, 16)]
            plsc.store_scatter(stage, [rows, lp], zero16)
            return cc

        lax.fori_loop(0, CHUNK // 16, cbody, 0)

    def compute_and_fire(b):
        _, _, dst_r, dsh, lpos, dst_s, dsh_s, q_rows, kv_rows, vrow, \
            stage, _, _, semv, sems = bufs[b]

        def gbody(gg, cc):
            rows = lane + gg * 16
            zi16 = jnp.zeros((16,), jnp.int32)
            f32 = jnp.float32

            def unpk(p):
                lo = lax.bitcast_convert_type(p << 16, f32)
                hi = lax.bitcast_convert_type(p & (-65536), f32)
                return lo, hi

            def dot_step(j, car):
                dvec, cj, cq = car
                kl, kh = unpk(plsc.load_gather(kv_rows, [rows, cj]))
                q0 = plsc.load_gather(q_rows, [rows, cq])
                q1 = plsc.load_gather(q_rows, [rows, cq + 1])
                return (dvec + q0 * kl + q1 * kh, cj + 1, cq + 2)

            dvec, _, _ = lax.fori_loop(0, HID // 2, dot_step,
                                       (zero16, zi16, zi16), unroll=16)
            x = dvec * inv_sqrt
            sg = 1.0 / (1.0 + jnp.exp(-x))
            svec = jnp.exp(sg)

            def sc_step(j, car):
                cv, cj = car
                vl, vh = unpk(plsc.load_gather(kv_rows, [rows, cv]))
                plsc.store_scatter(vrow, [rows, cj], vl * svec)
                plsc.store_scatter(vrow, [rows, cj + 1], vh * svec)
                return (cv + 1, cj + 2)

            lax.fori_loop(0, HID // 2, sc_step,
                          (jnp.full((16,), HID // 2, jnp.int32), zi16),
                          unroll=16)
            dst_g = dst_r[pl.ds(gg * 16, 16)]
            lp = lax.bitwise_and(dst_g, c127)
            lpos[pl.ds(gg * 16, 16)] = lp
            plsc.store_scatter(stage, [rows, lp], svec)
            return cc

        lax.fori_loop(0, CHUNK // 16, gbody, 0)
        for j in range(CHUNK // 16):
            sl = pl.ds(16 * j, 16)
            dst_s[sl] = dst_r[sl]
            dsh_s[sl] = dsh[sl]
        pltpu.async_copy(vrow, t_sh.at[dst_s], semv, add=True)
        pltpu.async_copy(stage, den_sh.at[dsh_s], sems, add=True)

    load_super(jnp.int32(0))
    prep_and_fire(jnp.int32(0), 0)

    def pair_body(t, carry):
        for half in range(2):
            g = t * 2 + half
            b = half
            gn = g + 1
            load_super(gn)
            prep_and_fire(gn, 1 - b)

            @pl.when(t >= 1)
            def _():
                wait_scatters(b)
                clear_stage(b)

            wait_gathers(b)
            compute_and_fire(b)
        return carry

    lax.fori_loop(0, (CHUNKS_PER_TILE - 1) // 2, pair_body, 0)

    wait_scatters(0)
    clear_stage(0)
    wait_gathers(0)
    compute_and_fire(0)

    wait_scatters(1)
    wait_scatters(0)
    plsc.subcore_barrier()

    ob = c * N_NODES + s * OUT_ROWS
    pltpu.sync_copy(t_sh.at[pl.ds(s * OUT_ROWS, OUT_ROWS)],
                    t_out_ref.at[pl.ds(ob, OUT_ROWS)])

    @pl.when(s == N_TILES - 1)
    def _():
        pltpu.sync_copy(t_sh.at[pl.ds(LAST_BASE, LAST_EXTRA)],
                        t_out_ref.at[pl.ds(c * N_NODES + LAST_BASE,
                                           LAST_EXTRA)])

    @pl.when(s < DEN_ROWS // 8)
    def _():
        pltpu.sync_copy(den_sh.at[pl.ds(s * 8, 8)],
                        den_out_ref.at[pl.ds(c * DEN_ROWS + s * 8, 8)])


def _run_edge(qt2, kv2, src_e, dst_e, zrow):
    f32 = jnp.float32
    mesh = plsc.VectorSubcoreMesh(core_axis_name="c", subcore_axis_name="s")
    kfn = functools.partial(
        pl.kernel,
        mesh=mesh,
        compiler_params=pltpu.CompilerParams(needs_layout_passes=False),
        out_type=[
            jax.ShapeDtypeStruct((H * N_NODES, HID), f32),
            jax.ShapeDtypeStruct((H * DEN_ROWS, HID), f32),
        ],
        scratch_types=(
            [pltpu.VMEM((SUPER,), jnp.int32)] * 2
            + [pltpu.VMEM((CHUNK,), jnp.int32)] * 14
            + [pltpu.VMEM((CHUNK, HID), f32),
               pltpu.VMEM((CHUNK, HID), f32),
               pltpu.VMEM((CHUNK, HID), jnp.int32),
               pltpu.VMEM((CHUNK, HID), jnp.int32),
               pltpu.VMEM((CHUNK, HID), f32),
               pltpu.VMEM((CHUNK, HID), f32),
               pltpu.VMEM((CHUNK, HID), f32),
               pltpu.VMEM((CHUNK, HID), f32),
               pltpu.VMEM_SHARED((N_NODES, HID), f32),
               pltpu.VMEM_SHARED((DEN_ROWS, HID), f32)]
            + [pltpu.SemaphoreType.DMA] * 8
        ),
    )
    return kfn(_edge_body)(qt2, kv2, src_e, dst_e, zrow)


def _out_body(t_ref, den_ref, hd_ref, wa_ref, ba_ref, out_ref):
    acc = None
    for h in range(H):
        den = den_ref[h]
        den = jnp.where(den == 0.0, np.float32(1.0), den)
        trans = t_ref[h] / den + hd_ref[h]
        o = jnp.dot(trans, wa_ref[h], preferred_element_type=jnp.float32) \
            + ba_ref[h]
        acc = o if acc is None else acc + o
    out_ref[...] = np.float32(1.0 / H) * acc


def _run_out(t, den, hd, Wa, ba):
    grid = N_NODES // ROW_BLK
    f32 = jnp.float32
    wfull = lambda s: pl.BlockSpec(s, lambda i: tuple(0 for _ in s))
    return pl.pallas_call(
        _out_body,
        grid=(grid,),
        in_specs=[
            pl.BlockSpec((H, ROW_BLK, HID), lambda i: (0, i, 0)),
            pl.BlockSpec((H, ROW_BLK, 1), lambda i: (0, i, 0)),
            pl.BlockSpec((H, ROW_BLK, HID), lambda i: (0, i, 0)),
            wfull(Wa.shape),
            wfull(ba.shape),
        ],
        out_specs=pl.BlockSpec((ROW_BLK, D), lambda i: (i, 0)),
        out_shape=jax.ShapeDtypeStruct((N_NODES, D), f32),
    )(t, den, hd, Wa, ba)


def kernel(src_x, dst_x, edge_index, W_src, b_src, g_src, be_src,
           W_dst, b_dst, g_dst, be_dst, Wk, bk, Wv, bv, Wq, bq, Wa, ba):
    src_e = edge_index[0].astype(jnp.int32)
    dst_e = edge_index[1].astype(jnp.int32)

    cs, ms, cd, md = _run_stats(src_x, dst_x)
    wps, bps, wpd, bpd = _run_fold(cs, ms, cd, md,
                                   W_src, b_src, g_src, be_src,
                                   W_dst, b_dst, g_dst, be_dst)
    qt, kv, hd = _run_proj(src_x, dst_x, wps, bps, wpd, bpd,
                           Wk, bk, Wv, bv, Wq, bq)
    qt2 = qt.reshape(H * N_NODES, HID)
    kv2 = lax.bitcast_convert_type(
        kv.reshape(H * N_NODES, HID, 2), jnp.int32)
    zrow = jnp.zeros((OUT_ROWS, HID), jnp.float32)
    t_out, den_out = _run_edge(qt2, kv2, src_e, dst_e, zrow)
    t = t_out.reshape(H, N_NODES, HID)
    den = den_out.reshape(H, DEN_ROWS * HID)[:, :N_NODES]
    den = den.reshape(H, N_NODES, 1)
    return _run_out(t, den, hd, Wa, ba)


# speedup vs baseline: 3.6534x; 1.0035x over previous
import functools

import jax
import jax.numpy as jnp
import numpy as np
from jax import lax
from jax.experimental import pallas as pl
from jax.experimental.pallas import tpu as pltpu
from jax.experimental.pallas import tpu_sc as plsc

N_NODES = 10000
E_TOTAL = 320000
D = 128
HID = 128
H = 2
EPS = 1e-5

ROW_BLK = 2000
N_TILES = 16
CHUNK = 32
EDGES_PER_TILE = E_TOTAL // N_TILES
CHUNKS_PER_TILE = EDGES_PER_TILE // CHUNK
SUPER = 800
CH_PER_SUP = SUPER // CHUNK
OUT_ROWS = 624
LAST_BASE = OUT_ROWS * N_TILES
LAST_EXTRA = N_NODES - LAST_BASE
DEN_ROWS = 80


def _stats_body(xs_ref, xd_ref, cs_ref, ms_ref, cd_ref, md_ref):
    @pl.when(pl.program_id(0) == 0)
    def _():
        cs_ref[...] = jnp.zeros_like(cs_ref)
        ms_ref[...] = jnp.zeros_like(ms_ref)
        cd_ref[...] = jnp.zeros_like(cd_ref)
        md_ref[...] = jnp.zeros_like(md_ref)

    xs = xs_ref[...]
    xd = xd_ref[...]
    dn = (((0,), (0,)), ((), ()))
    cs_ref[...] += lax.dot_general(xs, xs, dn, preferred_element_type=jnp.float32)
    ms_ref[...] += jnp.sum(xs, axis=0, keepdims=True)
    cd_ref[...] += lax.dot_general(xd, xd, dn, preferred_element_type=jnp.float32)
    md_ref[...] += jnp.sum(xd, axis=0, keepdims=True)


def _run_stats(src_x, dst_x):
    grid = N_NODES // ROW_BLK
    f32 = jnp.float32
    return pl.pallas_call(
        _stats_body,
        grid=(grid,),
        in_specs=[
            pl.BlockSpec((ROW_BLK, D), lambda i: (i, 0)),
            pl.BlockSpec((ROW_BLK, D), lambda i: (i, 0)),
        ],
        out_specs=[
            pl.BlockSpec((D, D), lambda i: (0, 0)),
            pl.BlockSpec((1, D), lambda i: (0, 0)),
            pl.BlockSpec((D, D), lambda i: (0, 0)),
            pl.BlockSpec((1, D), lambda i: (0, 0)),
        ],
        out_shape=[
            jax.ShapeDtypeStruct((D, D), f32),
            jax.ShapeDtypeStruct((1, D), f32),
            jax.ShapeDtypeStruct((D, D), f32),
            jax.ShapeDtypeStruct((1, D), f32),
        ],
    )(src_x, dst_x)


def _fold_body(cs_ref, ms_ref, cd_ref, md_ref,
               ws_ref, bs_ref, gs_ref, es_ref,
               wd_ref, bd_ref, gd_ref, ed_ref,
               wps_ref, bps_ref, wpd_ref, bpd_ref):
    inv_n = np.float32(1.0 / N_NODES)

    def fold(c_ref, m_ref, w_ref, b_ref, g_ref, e_ref, wo_ref, bo_ref):
        C = c_ref[...]
        m = m_ref[...] * inv_n
        for h in range(H):
            W = w_ref[h]
            g = g_ref[h : h + 1, :]
            be = e_ref[h : h + 1, :]
            mw = jnp.dot(m, W, preferred_element_type=jnp.float32)
            cw = jnp.dot(C, W, preferred_element_type=jnp.float32)
            ex2 = jnp.sum(W * cw, axis=0, keepdims=True) * inv_n
            var = ex2 - mw * mw
            scale = g * lax.rsqrt(var + np.float32(EPS))
            wo_ref[h] = W * scale
            bo_ref[h : h + 1, :] = be - mw * scale

    fold(cs_ref, ms_ref, ws_ref, bs_ref, gs_ref, es_ref, wps_ref, bps_ref)
    fold(cd_ref, md_ref, wd_ref, bd_ref, gd_ref, ed_ref, wpd_ref, bpd_ref)


def _run_fold(cs, ms, cd, md, W_src, b_src, g_src, be_src,
              W_dst, b_dst, g_dst, be_dst):
    f32 = jnp.float32
    full = lambda s: pl.BlockSpec(s, lambda: tuple(0 for _ in s))
    ins = [cs, ms, cd, md, W_src, b_src, g_src, be_src,
           W_dst, b_dst, g_dst, be_dst]
    return pl.pallas_call(
        _fold_body,
        in_specs=[full(x.shape) for x in ins],
        out_specs=[full((H, D, HID)), full((H, HID)),
                   full((H, D, HID)), full((H, HID))],
        out_shape=[
            jax.ShapeDtypeStruct((H, D, HID), f32),
            jax.ShapeDtypeStruct((H, HID), f32),
            jax.ShapeDtypeStruct((H, D, HID), f32),
            jax.ShapeDtypeStruct((H, HID), f32),
        ],
    )(*ins)


def _proj_body(xs_ref, xd_ref, wps_ref, bps_ref, wpd_ref, bpd_ref,
               wk_ref, bk_ref, wv_ref, bv_ref, wq_ref, bq_ref,
               qt_ref, kv_ref, hd_ref):
    xs = xs_ref[...]
    xd = xd_ref[...]
    for h in range(H):
        hs = jnp.maximum(
            jnp.dot(xs, wps_ref[h], preferred_element_type=jnp.float32)
            + bps_ref[h], 0.0)
        hd = jnp.maximum(
            jnp.dot(xd, wpd_ref[h], preferred_element_type=jnp.float32)
            + bpd_ref[h], 0.0)
        k = jnp.dot(hs, wk_ref[h], preferred_element_type=jnp.float32) + bk_ref[h]
        v = jnp.dot(hs, wv_ref[h], preferred_element_type=jnp.float32) + bv_ref[h]
        q = jnp.dot(hd, wq_ref[h], preferred_element_type=jnp.float32) + bq_ref[h]
        qt_ref[h] = q
        kv_ref[h] = jnp.concatenate([k, v], axis=1).astype(jnp.bfloat16)
        hd_ref[h] = hd


def _run_proj(src_x, dst_x, wps, bps, wpd, bpd, Wk, bk, Wv, bv, Wq, bq):
    grid = N_NODES // ROW_BLK
    f32 = jnp.float32
    wfull = lambda s: pl.BlockSpec(s, lambda i: tuple(0 for _ in s))
    ins = [wps, bps, wpd, bpd, Wk, bk, Wv, bv, Wq, bq]
    return pl.pallas_call(
        _proj_body,
        grid=(grid,),
        in_specs=[pl.BlockSpec((ROW_BLK, D), lambda i: (i, 0)),
                  pl.BlockSpec((ROW_BLK, D), lambda i: (i, 0))]
                 + [wfull(x.shape) for x in ins],
        out_specs=[
            pl.BlockSpec((H, ROW_BLK, HID), lambda i: (0, i, 0)),
            pl.BlockSpec((H, ROW_BLK, 2 * HID), lambda i: (0, i, 0)),
            pl.BlockSpec((H, ROW_BLK, HID), lambda i: (0, i, 0)),
        ],
        out_shape=[
            jax.ShapeDtypeStruct((H, N_NODES, HID), f32),
            jax.ShapeDtypeStruct((H, N_NODES, 2 * HID), jnp.bfloat16),
            jax.ShapeDtypeStruct((H, N_NODES, HID), f32),
        ],
    )(src_x, dst_x, *ins)


def _edge_body(qt_ref, kv_ref, src_ref, dst_ref, zrow_ref,
               t_out_ref, den_out_ref,
               src_sup, dst_sup,
               src_a0, src_a1, dst_a0, dst_a1, dst_r0, dst_r1,
               dst_s0, dst_s1,
               q0, q1, kv0, kv1, vr0, vr1, den_loc, iden,
               t_sh, den_sh,
               semq0, semq1, semk0, semk1, semv0, semv1):
    c = lax.axis_index("c")
    s = lax.axis_index("s")
    off = c * N_NODES
    tile_base = s * EDGES_PER_TILE
    inv_sqrt = np.float32(1.0 / np.sqrt(HID))
    lane = lax.iota(jnp.int32, 16)
    zero16 = jnp.zeros((16,), jnp.float32)
    c127 = jnp.full((16,), 127, jnp.int32)

    bufs = (
        (src_a0, dst_a0, dst_r0, dst_s0, q0, kv0, vr0,
         semq0, semk0, semv0),
        (src_a1, dst_a1, dst_r1, dst_s1, q1, kv1, vr1,
         semq1, semk1, semv1),
    )

    pltpu.sync_copy(zrow_ref.at[pl.ds(0, OUT_ROWS)],
                    t_sh.at[pl.ds(s * OUT_ROWS, OUT_ROWS)])

    @pl.when(s == N_TILES - 1)
    def _():
        pltpu.sync_copy(zrow_ref.at[pl.ds(0, LAST_EXTRA)],
                        t_sh.at[pl.ds(LAST_BASE, LAST_EXTRA)])

    @pl.when(s == 0)
    def _():
        pltpu.sync_copy(zrow_ref.at[pl.ds(0, DEN_ROWS)], den_sh)

    def zloc(i, cc):
        for j in range(HID // 16):
            den_loc[i, pl.ds(16 * j, 16)] = zero16
        return cc

    lax.fori_loop(0, DEN_ROWS, zloc, 0)
    for j in range(DEN_ROWS // 16):
        iden[pl.ds(16 * j, 16)] = lane + 16 * j

    plsc.subcore_barrier()

    def load_super(gn):
        @pl.when(lax.rem(gn, CH_PER_SUP) == 0)
        def _():
            base = tile_base + gn * CHUNK
            pltpu.sync_copy(src_ref.at[pl.ds(base, SUPER)], src_sup)
            pltpu.sync_copy(dst_ref.at[pl.ds(base, SUPER)], dst_sup)

    def prep_and_fire(gn, b):
        src_a, dst_a, dst_r, _, q_rows, kv_rows, _, semq, semk, _ = bufs[b]
        pos = lax.rem(gn, CH_PER_SUP) * CHUNK
        for j in range(CHUNK // 16):
            sl = pl.ds(pos + 16 * j, 16)
            ob = pl.ds(16 * j, 16)
            sv = src_sup[sl]
            dv = dst_sup[sl]
            src_a[ob] = sv + off
            dst_a[ob] = dv + off
            dst_r[ob] = dv
        pltpu.async_copy(qt_ref.at[dst_a], q_rows, semq)
        pltpu.async_copy(kv_ref.at[src_a], kv_rows, semk)

    def wait_gathers(b):
        src_a, dst_a, _, _, q_rows, kv_rows, _, semq, semk, _ = bufs[b]
        pltpu.make_async_copy(qt_ref.at[dst_a], q_rows, semq).wait()
        pltpu.make_async_copy(kv_ref.at[src_a], kv_rows, semk).wait()

    def wait_scatters(b):
        _, _, _, dst_s, _, _, vrow, _, _, semv = bufs[b]
        pltpu.make_async_copy(vrow, t_sh.at[dst_s], semv).wait()

    def compute_and_fire(b):
        _, _, dst_r, dst_s, q_rows, kv_rows, vrow, _, _, semv = bufs[b]

        def gbody(gg, cc):
            rows = lane + gg * 16
            zi16 = jnp.zeros((16,), jnp.int32)
            f32 = jnp.float32

            def unpk(p):
                lo = lax.bitcast_convert_type(p << 16, f32)
                hi = lax.bitcast_convert_type(p & (-65536), f32)
                return lo, hi

            def dot_step(j, car):
                dvec, cj, cq = car
                kl, kh = unpk(plsc.load_gather(kv_rows, [rows, cj]))
                q0 = plsc.load_gather(q_rows, [rows, cq])
                q1 = plsc.load_gather(q_rows, [rows, cq + 1])
                return (dvec + q0 * kl + q1 * kh, cj + 1, cq + 2)

            dvec, _, _ = lax.fori_loop(0, HID // 2, dot_step,
                                       (zero16, zi16, zi16), unroll=16)
            x = dvec * inv_sqrt
            sg = 1.0 / (1.0 + jnp.exp(-x))
            svec = jnp.exp(sg)

            def sc_step(j, car):
                cv, cj = car
                vl, vh = unpk(plsc.load_gather(kv_rows, [rows, cv]))
                plsc.store_scatter(vrow, [rows, cj], vl * svec)
                plsc.store_scatter(vrow, [rows, cj + 1], vh * svec)
                return (cv + 1, cj + 2)

            lax.fori_loop(0, HID // 2, sc_step,
                          (jnp.full((16,), HID // 2, jnp.int32), zi16),
                          unroll=16)
            dst_g = dst_r[pl.ds(gg * 16, 16)]
            rowv = lax.shift_right_logical(dst_g, 7)
            lp = lax.bitwise_and(dst_g, c127)
            plsc.addupdate_scatter(den_loc, [rowv, lp], svec)
            return cc

        lax.fori_loop(0, CHUNK // 16, gbody, 0)
        for j in range(CHUNK // 16):
            sl = pl.ds(16 * j, 16)
            dst_s[sl] = dst_r[sl]
        pltpu.async_copy(vrow, t_sh.at[dst_s], semv, add=True)

    load_super(jnp.int32(0))
    prep_and_fire(jnp.int32(0), 0)

    def pair_body(t, carry):
        for half in range(2):
            g = t * 2 + half
            b = half
            gn = g + 1
            load_super(gn)
            prep_and_fire(gn, 1 - b)

            @pl.when(t >= 1)
            def _():
                wait_scatters(b)

            wait_gathers(b)
            compute_and_fire(b)
        return carry

    lax.fori_loop(0, (CHUNKS_PER_TILE - 1) // 2, pair_body, 0)

    wait_scatters(0)
    wait_gathers(0)
    compute_and_fire(0)

    wait_scatters(1)
    wait_scatters(0)
    pltpu.sync_copy(den_loc, den_sh.at[iden], add=True)
    plsc.subcore_barrier()

    ob = c * N_NODES + s * OUT_ROWS
    pltpu.sync_copy(t_sh.at[pl.ds(s * OUT_ROWS, OUT_ROWS)],
                    t_out_ref.at[pl.ds(ob, OUT_ROWS)])

    @pl.when(s == N_TILES - 1)
    def _():
        pltpu.sync_copy(t_sh.at[pl.ds(LAST_BASE, LAST_EXTRA)],
                        t_out_ref.at[pl.ds(c * N_NODES + LAST_BASE,
                                           LAST_EXTRA)])

    @pl.when(s < DEN_ROWS // 8)
    def _():
        pltpu.sync_copy(den_sh.at[pl.ds(s * 8, 8)],
                        den_out_ref.at[pl.ds(c * DEN_ROWS + s * 8, 8)])


def _run_edge(qt2, kv2, src_e, dst_e, zrow):
    f32 = jnp.float32
    mesh = plsc.VectorSubcoreMesh(core_axis_name="c", subcore_axis_name="s")
    kfn = functools.partial(
        pl.kernel,
        mesh=mesh,
        compiler_params=pltpu.CompilerParams(needs_layout_passes=False),
        out_type=[
            jax.ShapeDtypeStruct((H * N_NODES, HID), f32),
            jax.ShapeDtypeStruct((H * DEN_ROWS, HID), f32),
        ],
        scratch_types=(
            [pltpu.VMEM((SUPER,), jnp.int32)] * 2
            + [pltpu.VMEM((CHUNK,), jnp.int32)] * 8
            + [pltpu.VMEM((CHUNK, HID), f32),
               pltpu.VMEM((CHUNK, HID), f32),
               pltpu.VMEM((CHUNK, HID), jnp.int32),
               pltpu.VMEM((CHUNK, HID), jnp.int32),
               pltpu.VMEM((CHUNK, HID), f32),
               pltpu.VMEM((CHUNK, HID), f32),
               pltpu.VMEM((DEN_ROWS, HID), f32),
               pltpu.VMEM((DEN_ROWS,), jnp.int32),
               pltpu.VMEM_SHARED((N_NODES, HID), f32),
               pltpu.VMEM_SHARED((DEN_ROWS, HID), f32)]
            + [pltpu.SemaphoreType.DMA] * 6
        ),
    )
    return kfn(_edge_body)(qt2, kv2, src_e, dst_e, zrow)


def _out_body(t_ref, den_ref, hd_ref, wa_ref, ba_ref, out_ref):
    acc = None
    for h in range(H):
        den = den_ref[h]
        den = jnp.where(den == 0.0, np.float32(1.0), den)
        trans = t_ref[h] / den + hd_ref[h]
        o = jnp.dot(trans, wa_ref[h], preferred_element_type=jnp.float32) \
            + ba_ref[h]
        acc = o if acc is None else acc + o
    out_ref[...] = np.float32(1.0 / H) * acc


def _run_out(t, den, hd, Wa, ba):
    grid = N_NODES // ROW_BLK
    f32 = jnp.float32
    wfull = lambda s: pl.BlockSpec(s, lambda i: tuple(0 for _ in s))
    return pl.pallas_call(
        _out_body,
        grid=(grid,),
        in_specs=[
            pl.BlockSpec((H, ROW_BLK, HID), lambda i: (0, i, 0)),
            pl.BlockSpec((H, ROW_BLK, 1), lambda i: (0, i, 0)),
            pl.BlockSpec((H, ROW_BLK, HID), lambda i: (0, i, 0)),
            wfull(Wa.shape),
            wfull(ba.shape),
        ],
        out_specs=pl.BlockSpec((ROW_BLK, D), lambda i: (i, 0)),
        out_shape=jax.ShapeDtypeStruct((N_NODES, D), f32),
    )(t, den, hd, Wa, ba)


def kernel(src_x, dst_x, edge_index, W_src, b_src, g_src, be_src,
           W_dst, b_dst, g_dst, be_dst, Wk, bk, Wv, bv, Wq, bq, Wa, ba):
    src_e = edge_index[0].astype(jnp.int32)
    dst_e = edge_index[1].astype(jnp.int32)

    cs, ms, cd, md = _run_stats(src_x, dst_x)
    wps, bps, wpd, bpd = _run_fold(cs, ms, cd, md,
                                   W_src, b_src, g_src, be_src,
                                   W_dst, b_dst, g_dst, be_dst)
    qt, kv, hd = _run_proj(src_x, dst_x, wps, bps, wpd, bpd,
                           Wk, bk, Wv, bv, Wq, bq)
    qt2 = qt.reshape(H * N_NODES, HID)
    kv2 = lax.bitcast_convert_type(
        kv.reshape(H * N_NODES, HID, 2), jnp.int32)
    zrow = jnp.zeros((OUT_ROWS, HID), jnp.float32)
    t_out, den_out = _run_edge(qt2, kv2, src_e, dst_e, zrow)
    t = t_out.reshape(H, N_NODES, HID)
    den = den_out.reshape(H, DEN_ROWS * HID)[:, :N_NODES]
    den = den.reshape(H, N_NODES, 1)
    return _run_out(t, den, hd, Wa, ba)
